# Initial kernel scaffold; baseline (speedup 1.0000x reference)
import jax
import jax.numpy as jnp
from jax.experimental import pallas as pl


def kernel(X, W1, W2):
    raise NotImplementedError("write your pallas kernel here")


# speedup vs baseline: 2245.0069x; 2245.0069x over previous
import jax
import jax.numpy as jnp
from jax.experimental import pallas as pl

_NT = 1024
_D = 32


def _elu(x):
    return jnp.where(x > 0, x, jnp.exp(x) - 1.0)


def _fused_gcn_kernel(x_ref, w1_ref, w2_ref, o_ref):
    x = x_ref[0]
    a = jnp.dot(x, x.T, preferred_element_type=jnp.float32)
    a = jnp.tanh(jax.nn.relu(a))
    h = x
    for w_ref in (w1_ref, w2_ref):
        agg = jnp.dot(a, h, preferred_element_type=jnp.float32)
        h = _elu(jnp.dot(agg, w_ref[...], preferred_element_type=jnp.float32))
    o_ref[0] = h


def kernel(X, W1, W2):
    Bv, NTv, Dv = X.shape
    out = pl.pallas_call(
        _fused_gcn_kernel,
        grid=(Bv,),
        in_specs=[
            pl.BlockSpec((1, NTv, Dv), lambda b: (b, 0, 0)),
            pl.BlockSpec((Dv, Dv), lambda b: (0, 0)),
            pl.BlockSpec((Dv, Dv), lambda b: (0, 0)),
        ],
        out_specs=pl.BlockSpec((1, NTv, Dv), lambda b: (b, 0, 0)),
        out_shape=jax.ShapeDtypeStruct((Bv, NTv, Dv), jnp.float32),
    )(X, W1, W2)
    return out
